# baseline (speedup 1.0000x reference)
import jax
import jax.numpy as jnp
from jax.experimental import pallas as pl
from jax.experimental.pallas import tpu as pltpu

TOP_K = 8
NUM_EXPERTS = 64
BLOCK_TOKENS = 2048
NEG_INF = float("-inf")


def _topk_kernel(x_ref, w_ref, id_ref, id2_ref):
    xt = x_ref[:, :].T
    iota_f = jax.lax.broadcasted_iota(jnp.int32, xt.shape, 0).astype(jnp.float32)
    ws = []
    ids = []
    for _ in range(TOP_K):
        cur = jnp.max(xt, axis=0, keepdims=True)
        hit = xt == cur
        idx = jnp.min(jnp.where(hit, iota_f, 64.0), axis=0, keepdims=True)
        ws.append(cur)
        ids.append(idx)
        xt = jnp.where(iota_f == idx, NEG_INF, xt)
    w = jnp.concatenate(ws, axis=0)
    w = jnp.exp(w - w[:1, :])
    w = w / jnp.sum(w, axis=0, keepdims=True)
    w_ref[:, :] = w.T
    idi = jnp.concatenate(ids, axis=0).T.astype(jnp.int32)
    id_ref[:, :] = idi
    id2_ref[:, :] = idi


def kernel(router_logits_fp32, topk_ids, topk_weights):
    del topk_ids, topk_weights
    n, _ = router_logits_fp32.shape
    grid = (n // BLOCK_TOKENS,)
    w, ids, ids2 = pl.pallas_call(
        _topk_kernel,
        grid=grid,
        in_specs=[pl.BlockSpec((BLOCK_TOKENS, NUM_EXPERTS), lambda i: (i, 0))],
        out_specs=[
            pl.BlockSpec((BLOCK_TOKENS, TOP_K), lambda i: (i, 0)),
            pl.BlockSpec((BLOCK_TOKENS, TOP_K), lambda i: (i, 0)),
            pl.BlockSpec((BLOCK_TOKENS, TOP_K), lambda i: (i, 0)),
        ],
        out_shape=[
            jax.ShapeDtypeStruct((n, TOP_K), jnp.float32),
            jax.ShapeDtypeStruct((n, TOP_K), jnp.int32),
            jax.ShapeDtypeStruct((n, TOP_K), jnp.int32),
        ],
        compiler_params=pltpu.CompilerParams(
            dimension_semantics=("parallel",),
        ),
    )(router_logits_fp32)
    return (w, ids, ids2)


# speedup vs baseline: 1.1748x; 1.1748x over previous
import jax
import jax.numpy as jnp
from jax.experimental import pallas as pl
from jax.experimental.pallas import tpu as pltpu

TOP_K = 8
NUM_EXPERTS = 64
BLOCK_TOKENS = 2048
NEG_INF = float("-inf")


def _topk_kernel(x_ref, w_ref, id_ref):
    xt = x_ref[:, :].T
    iota_f = jax.lax.broadcasted_iota(jnp.int32, xt.shape, 0).astype(jnp.float32)
    ws = []
    ids = []
    for _ in range(TOP_K):
        cur = jnp.max(xt, axis=0, keepdims=True)
        hit = xt == cur
        idx = jnp.min(jnp.where(hit, iota_f, 64.0), axis=0, keepdims=True)
        ws.append(cur)
        ids.append(idx)
        xt = jnp.where(iota_f == idx, NEG_INF, xt)
    w = jnp.concatenate(ws, axis=0)
    w = jnp.exp(w - w[:1, :])
    w = w / jnp.sum(w, axis=0, keepdims=True)
    w_ref[:, :] = w.T
    id_ref[:, :] = jnp.concatenate(ids, axis=0).T.astype(jnp.int32)


def kernel(router_logits_fp32, topk_ids, topk_weights):
    del topk_ids, topk_weights
    n, _ = router_logits_fp32.shape
    grid = (n // BLOCK_TOKENS,)
    w, ids = pl.pallas_call(
        _topk_kernel,
        grid=grid,
        in_specs=[pl.BlockSpec((BLOCK_TOKENS, NUM_EXPERTS), lambda i: (i, 0))],
        out_specs=[
            pl.BlockSpec((BLOCK_TOKENS, TOP_K), lambda i: (i, 0)),
            pl.BlockSpec((BLOCK_TOKENS, TOP_K), lambda i: (i, 0)),
        ],
        out_shape=[
            jax.ShapeDtypeStruct((n, TOP_K), jnp.float32),
            jax.ShapeDtypeStruct((n, TOP_K), jnp.int32),
        ],
        compiler_params=pltpu.CompilerParams(
            dimension_semantics=("parallel",),
        ),
    )(router_logits_fp32)
    return (w, ids, ids)


# speedup vs baseline: 1.2529x; 1.0665x over previous
import jax
import jax.numpy as jnp
from jax.experimental import pallas as pl
from jax.experimental.pallas import tpu as pltpu

TOP_K = 8
NUM_EXPERTS = 64
BLOCK_TOKENS = 4096
NEG_INF = float("-inf")


def _topk_kernel(x_ref, w_ref, id_ref):
    xt = x_ref[:, :].T
    iota_f = jax.lax.broadcasted_iota(jnp.int32, xt.shape, 0).astype(jnp.float32)
    ws = []
    ids = []
    for _ in range(TOP_K):
        cur = jnp.max(xt, axis=0, keepdims=True)
        hit = xt == cur
        idx = jnp.min(jnp.where(hit, iota_f, 64.0), axis=0, keepdims=True)
        ws.append(cur)
        ids.append(idx)
        xt = jnp.where(iota_f == idx, NEG_INF, xt)
    w = jnp.concatenate(ws, axis=0)
    w = jnp.exp(w - w[:1, :])
    w = w / jnp.sum(w, axis=0, keepdims=True)
    w_ref[:, :] = w.T
    id_ref[:, :] = jnp.concatenate(ids, axis=0).T.astype(jnp.int32)


def kernel(router_logits_fp32, topk_ids, topk_weights):
    del topk_ids, topk_weights
    n, _ = router_logits_fp32.shape
    grid = (n // BLOCK_TOKENS,)
    w, ids = pl.pallas_call(
        _topk_kernel,
        grid=grid,
        in_specs=[pl.BlockSpec((BLOCK_TOKENS, NUM_EXPERTS), lambda i: (i, 0))],
        out_specs=[
            pl.BlockSpec((BLOCK_TOKENS, TOP_K), lambda i: (i, 0)),
            pl.BlockSpec((BLOCK_TOKENS, TOP_K), lambda i: (i, 0)),
        ],
        out_shape=[
            jax.ShapeDtypeStruct((n, TOP_K), jnp.float32),
            jax.ShapeDtypeStruct((n, TOP_K), jnp.int32),
        ],
        compiler_params=pltpu.CompilerParams(
            dimension_semantics=("parallel",),
        ),
    )(router_logits_fp32)
    return (w, ids, ids)
